# baseline (speedup 1.0000x reference)
import functools

import jax
import jax.numpy as jnp
from jax import lax
from jax.experimental import pallas as pl
from jax.experimental.pallas import tpu as pltpu
from jax.experimental.pallas import tpu_sc as plsc

B, N = 128, 32768
T = B * N
NC, NS = 2, 16
NW = NC * NS
S = T // NW
C = 8192
NCHUNK = S // C
L = 16


def _apply_op(op, a, b):
    if op == 0:
        return a + b
    if op == 1:
        return a - b
    if op == 2:
        return a * b
    return a / (b + 1e-06)


def _node_pass(a_ref, b_ref, o_ref, s):
    for op in range(4):
        @pl.when(s == op)
        def _():
            def body(i, carry):
                sl = pl.ds(i * L, L)
                o_ref[sl] = _apply_op(op, a_ref[sl], b_ref[sl])
                return carry
            lax.fori_loop(0, C // L, body, 0)


def _sc_body(g1, g2, g3, g4, sel, out,
             a10, a20, a30, a40, a11, a21, a31, a41,
             o0, o1, selv,
             sin0, sin1, sout0, sout1):
    ins = ((a10, a20, a30, a40), (a11, a21, a31, a41))
    outs = (o0, o1)
    sem_in = (sin0, sin1)
    sem_out = (sout0, sout1)
    srcs = (g1, g2, g3, g4)

    wid = lax.axis_index("s") * NC + lax.axis_index("c")
    base = wid * S

    pltpu.sync_copy(sel, selv)
    s1 = selv[pl.ds(0, L)][0]
    s2 = selv[pl.ds(L, L)][0]
    s3 = selv[pl.ds(2 * L, L)][0]

    def start_in(b, chunk):
        off = base + chunk * C
        for g, dst in zip(srcs, ins[b]):
            pltpu.async_copy(g.at[pl.ds(off, C)], dst, sem_in[b])

    def wait_in(b, chunk):
        off = base + chunk * C
        for g, dst in zip(srcs, ins[b]):
            pltpu.make_async_copy(g.at[pl.ds(off, C)], dst, sem_in[b]).wait()

    start_in(0, 0)

    def chunk_step(b, chunk):
        wait_in(b, chunk)

        @pl.when(chunk + 1 < NCHUNK)
        def _():
            start_in(1 - b, chunk + 1)

        @pl.when(chunk >= 2)
        def _():
            off_prev = base + (chunk - 2) * C
            pltpu.make_async_copy(
                outs[b], out.at[pl.ds(off_prev, C)], sem_out[b]).wait()

        g1b, g2b, g3b, g4b = ins[b]
        _node_pass(g1b, g2b, g1b, s1)
        _node_pass(g3b, g4b, g3b, s2)
        _node_pass(g1b, g3b, outs[b], s3)

        off = base + chunk * C
        pltpu.async_copy(outs[b], out.at[pl.ds(off, C)], sem_out[b])

    def outer(j, carry):
        chunk_step(0, 2 * j)
        chunk_step(1, 2 * j + 1)
        return carry
    lax.fori_loop(0, NCHUNK // 2, outer, 0)

    for b, chunk in ((0, NCHUNK - 2), (1, NCHUNK - 1)):
        off = base + chunk * C
        pltpu.make_async_copy(
            outs[b], out.at[pl.ds(off, C)], sem_out[b]).wait()


_sc_fused = functools.partial(
    pl.kernel,
    out_type=jax.ShapeDtypeStruct((T,), jnp.float32),
    mesh=plsc.VectorSubcoreMesh(core_axis_name="c", subcore_axis_name="s"),
    scratch_types=(
        [pltpu.VMEM((C,), jnp.float32) for _ in range(10)]
        + [pltpu.VMEM((3 * L,), jnp.int32)]
        + [pltpu.SemaphoreType.DMA for _ in range(4)]
    ),
)(_sc_body)


def kernel(g1, g2, g3, g4, w1, w2, w3):
    root = jax.random.key(42)
    k1, k2, k3 = jax.random.split(root, 3)
    i11 = jax.random.categorical(k1, w1)
    i12 = jax.random.categorical(k2, w2)
    i21 = jax.random.categorical(k3, w3)
    idx = jnp.stack([i11, i12, i21])
    log_probs = jnp.stack([
        jax.nn.log_softmax(w1)[i11],
        jax.nn.log_softmax(w2)[i12],
        jax.nn.log_softmax(w3)[i21],
    ])

    sel = jnp.broadcast_to(
        idx.astype(jnp.int32)[:, None], (3, L)).reshape(3 * L)
    out = _sc_fused(
        g1.reshape(T), g2.reshape(T), g3.reshape(T), g4.reshape(T), sel)
    return (out.reshape(B, N), log_probs, idx)


# speedup vs baseline: 2.7018x; 2.7018x over previous
import functools

import jax
import jax.numpy as jnp
from jax import lax
from jax.experimental import pallas as pl
from jax.experimental.pallas import tpu as pltpu
from jax.experimental.pallas import tpu_sc as plsc

B, N = 128, 32768
NC, NS = 2, 16
NW = NC * NS
TR = B // 8
HALF = N // 2
CW = 1024
NCHUNK = HALF // CW
L = 16


def _node(a, b, mulm, sign, is_div):
    if is_div:
        return a / (b + 1e-06)
    return jnp.where(mulm, a * b, a + sign * b)


def _sc_body(g1, g2, g3, g4, aux, out,
             a10, a20, a30, a40, a11, a21, a31, a41,
             o0, o1, auxv,
             sin0, sin1, sout0, sout1):
    ins = ((a10, a20, a30, a40), (a11, a21, a31, a41))
    outs = (o0, o1)
    sem_in = (sin0, sin1)
    sem_out = (sout0, sout1)
    srcs = (g1, g2, g3, g4)

    wid = lax.axis_index("s") * NC + lax.axis_index("c")
    r0 = lax.rem(wid, TR) * 8
    c0 = lax.div(wid, TR) * HALF

    pltpu.sync_copy(aux, auxv)
    sel1 = auxv[0, pl.ds(0, L)]
    sel2 = auxv[1, pl.ds(0, L)]
    sel3 = auxv[2, pl.ds(0, L)]
    m1, m2, m3 = (s == 2 for s in (sel1, sel2, sel3))
    one = jnp.float32(1.0)
    sg1, sg2, sg3 = (jnp.where(s == 1, -one, one) for s in (sel1, sel2, sel3))
    d1, d2, d3 = (s[0] == 3 for s in (sel1, sel2, sel3))

    def start_in(b, chunk):
        c = c0 + chunk * CW
        for g, dst in zip(srcs, ins[b]):
            pltpu.async_copy(g.at[pl.ds(r0, 8), pl.ds(c, CW)], dst, sem_in[b])

    def wait_in(b, chunk):
        c = c0 + chunk * CW
        for g, dst in zip(srcs, ins[b]):
            pltpu.make_async_copy(
                g.at[pl.ds(r0, 8), pl.ds(c, CW)], dst, sem_in[b]).wait()

    start_in(0, 0)

    def compute(b, v1, v2, v3):
        g1b, g2b, g3b, g4b = ins[b]
        ob = outs[b]

        def body(i, carry):
            sl = pl.ds(i * L, L)
            for r in range(8):
                x1 = g1b[r, sl]
                x2 = g2b[r, sl]
                x3 = g3b[r, sl]
                x4 = g4b[r, sl]
                t1 = _node(x1, x2, m1, sg1, v1)
                t2 = _node(x3, x4, m2, sg2, v2)
                ob[r, sl] = _node(t1, t2, m3, sg3, v3)
            return carry
        lax.fori_loop(0, CW // L, body, 0)

    def chunk_step(b, chunk):
        wait_in(b, chunk)

        @pl.when(chunk + 1 < NCHUNK)
        def _():
            start_in(1 - b, chunk + 1)

        @pl.when(chunk >= 2)
        def _():
            c_prev = c0 + (chunk - 2) * CW
            pltpu.make_async_copy(
                outs[b], out.at[pl.ds(r0, 8), pl.ds(c_prev, CW)],
                sem_out[b]).wait()

        for v1 in (False, True):
            for v2 in (False, True):
                for v3 in (False, True):
                    cond = ((d1 == v1) & (d2 == v2) & (d3 == v3))
                    pl.when(cond)(
                        functools.partial(compute, b, v1, v2, v3))

        c = c0 + chunk * CW
        pltpu.async_copy(outs[b], out.at[pl.ds(r0, 8), pl.ds(c, CW)],
                         sem_out[b])

    def outer(j, carry):
        chunk_step(0, 2 * j)
        chunk_step(1, 2 * j + 1)
        return carry
    lax.fori_loop(0, NCHUNK // 2, outer, 0)

    for b, chunk in ((0, NCHUNK - 2), (1, NCHUNK - 1)):
        c = c0 + chunk * CW
        pltpu.make_async_copy(
            outs[b], out.at[pl.ds(r0, 8), pl.ds(c, CW)], sem_out[b]).wait()


_sc_fused = functools.partial(
    pl.kernel,
    out_type=jax.ShapeDtypeStruct((B, N), jnp.float32),
    mesh=plsc.VectorSubcoreMesh(core_axis_name="c", subcore_axis_name="s"),
    scratch_types=(
        [pltpu.VMEM((8, CW), jnp.float32) for _ in range(10)]
        + [pltpu.VMEM((8, 128), jnp.int32)]
        + [pltpu.SemaphoreType.DMA for _ in range(4)]
    ),
)(_sc_body)


def kernel(g1, g2, g3, g4, w1, w2, w3):
    root = jax.random.key(42)
    k1, k2, k3 = jax.random.split(root, 3)
    i11 = jax.random.categorical(k1, w1)
    i12 = jax.random.categorical(k2, w2)
    i21 = jax.random.categorical(k3, w3)
    idx = jnp.stack([i11, i12, i21])
    log_probs = jnp.stack([
        jax.nn.log_softmax(w1)[i11],
        jax.nn.log_softmax(w2)[i12],
        jax.nn.log_softmax(w3)[i21],
    ])

    aux = jnp.zeros((8, 128), jnp.int32).at[0:3, :].set(
        idx.astype(jnp.int32)[:, None])
    out = _sc_fused(g1, g2, g3, g4, aux)
    return (out, log_probs, idx)
